# baseline (speedup 1.0000x reference)
import jax
from jax import lax
import jax.numpy as jnp
from jax.experimental import pallas as pl


_NUM_CLASSES = 4
_LANES = 128
_ROW_BLK = 64


def _onehot_body(x_ref, o_ref):
    r, m = x_ref.shape
    jt = m // _LANES
    g = 2
    ci = lax.broadcasted_iota(
        jnp.int32, (r, g * _NUM_CLASSES, _LANES), 1
    ) % _NUM_CLASSES
    for k in range(jt // g):
        xc = x_ref[:, k * g * _LANES:(k + 1) * g * _LANES].reshape(r, g, _LANES)
        xrep = jnp.repeat(xc, _NUM_CLASSES, axis=1)
        o_ref[:, k * g * _NUM_CLASSES:(k + 1) * g * _NUM_CLASSES, :] = (
            xrep == ci
        ).astype(jnp.float32)


def kernel(x):
    n, m = x.shape
    jt = m // _LANES
    o = pl.pallas_call(
        _onehot_body,
        grid=(n // _ROW_BLK,),
        in_specs=[pl.BlockSpec((_ROW_BLK, m), lambda i: (i, 0))],
        out_specs=pl.BlockSpec(
            (_ROW_BLK, jt * _NUM_CLASSES, _LANES), lambda i: (i, 0, 0)
        ),
        out_shape=jax.ShapeDtypeStruct((n, jt * _NUM_CLASSES, _LANES), jnp.float32),
    )(x)
    return (
        o.reshape(n, jt, _NUM_CLASSES, _LANES)
        .transpose(0, 1, 3, 2)
        .reshape(n, m, _NUM_CLASSES)
    )


# speedup vs baseline: 1.0793x; 1.0793x over previous
import jax
from jax import lax
import jax.numpy as jnp
from jax.experimental import pallas as pl


_NUM_CLASSES = 4
_LANES = 128
_ROW_BLK = 256


def _onehot_body(x_ref, o_ref):
    r, m = x_ref.shape
    jt = m // _LANES
    g = 2
    ci = lax.broadcasted_iota(
        jnp.int32, (r, g * _NUM_CLASSES, _LANES), 1
    ) % _NUM_CLASSES
    for k in range(jt // g):
        xc = x_ref[:, k * g * _LANES:(k + 1) * g * _LANES].reshape(r, g, _LANES)
        xrep = jnp.repeat(xc, _NUM_CLASSES, axis=1)
        o_ref[:, k * g * _NUM_CLASSES:(k + 1) * g * _NUM_CLASSES, :] = (
            xrep == ci
        ).astype(jnp.float32)


def kernel(x):
    n, m = x.shape
    jt = m // _LANES
    o = pl.pallas_call(
        _onehot_body,
        grid=(n // _ROW_BLK,),
        in_specs=[pl.BlockSpec((_ROW_BLK, m), lambda i: (i, 0))],
        out_specs=pl.BlockSpec(
            (_ROW_BLK, jt * _NUM_CLASSES, _LANES), lambda i: (i, 0, 0)
        ),
        out_shape=jax.ShapeDtypeStruct((n, jt * _NUM_CLASSES, _LANES), jnp.float32),
    )(x)
    return (
        o.reshape(n, jt, _NUM_CLASSES, _LANES)
        .transpose(0, 1, 3, 2)
        .reshape(n, m, _NUM_CLASSES)
    )


# speedup vs baseline: 1.1137x; 1.0319x over previous
import jax
import jax.numpy as jnp
from jax import lax
from jax.experimental import pallas as pl
from jax.experimental.pallas import tpu as pltpu


_NUM_CLASSES = 4
_LANES = 128
_ROW_BLK = 128


def _onehot_body(x_hbm, o_ref, xbuf, sem):
    i = pl.program_id(0)
    nb = pl.num_programs(0)
    slot = lax.rem(i, 2)
    nxt = lax.rem(i + 1, 2)

    @pl.when(i == 0)
    def _():
        pltpu.make_async_copy(
            x_hbm.at[pl.ds(i * _ROW_BLK, _ROW_BLK)], xbuf.at[slot], sem.at[slot]
        ).start()

    @pl.when(i + 1 < nb)
    def _():
        pltpu.make_async_copy(
            x_hbm.at[pl.ds((i + 1) * _ROW_BLK, _ROW_BLK)],
            xbuf.at[nxt],
            sem.at[nxt],
        ).start()

    pltpu.make_async_copy(
        x_hbm.at[pl.ds(i * _ROW_BLK, _ROW_BLK)], xbuf.at[slot], sem.at[slot]
    ).wait()

    r = _ROW_BLK
    m = xbuf.shape[2]
    jt = m // _LANES
    g = 2
    ci = lax.broadcasted_iota(
        jnp.int32, (r, g * _NUM_CLASSES, _LANES), 1
    ) % _NUM_CLASSES
    for k in range(jt // g):
        xc = xbuf[slot, :, k * g * _LANES:(k + 1) * g * _LANES].reshape(
            r, g, _LANES
        )
        xrep = jnp.repeat(xc, _NUM_CLASSES, axis=1)
        o_ref[:, k * g * _NUM_CLASSES:(k + 1) * g * _NUM_CLASSES, :] = (
            xrep == ci
        ).astype(jnp.float32)


def kernel(x):
    n, m = x.shape
    jt = m // _LANES
    o = pl.pallas_call(
        _onehot_body,
        grid=(n // _ROW_BLK,),
        in_specs=[pl.BlockSpec(memory_space=pl.ANY)],
        out_specs=pl.BlockSpec(
            (_ROW_BLK, jt * _NUM_CLASSES, _LANES), lambda i: (i, 0, 0)
        ),
        out_shape=jax.ShapeDtypeStruct((n, jt * _NUM_CLASSES, _LANES), jnp.float32),
        scratch_shapes=[
            pltpu.VMEM((2, _ROW_BLK, m), jnp.int32),
            pltpu.SemaphoreType.DMA((2,)),
        ],
    )(x)
    return (
        o.reshape(n, jt, _NUM_CLASSES, _LANES)
        .transpose(0, 1, 3, 2)
        .reshape(n, m, _NUM_CLASSES)
    )
